# baseline (speedup 1.0000x reference)
import functools

import jax
import jax.numpy as jnp
from jax import lax
from jax.experimental import pallas as pl
from jax.experimental.pallas import tpu as pltpu
from jax.experimental.pallas import tpu_sc as plsc

N = 10000
E = 320000
D = 128
B = 4096
C = 64
NC = 2
NS = 16
NW = NC * NS
ECH = 80
EPT = 10240
EPAD = EPT * NW
NCHE = EPT // ECH
CGC = B // NW
NACC = 10240
RPT = NACC // NS
RB = 1000
EPS = 1e-5

_sc_cache = {}


def _sc_kernel(name, body, out_type, scratch_types):
    fn = _sc_cache.get(name)
    if fn is None:
        mesh = plsc.VectorSubcoreMesh(core_axis_name="c", subcore_axis_name="s",
                                      num_cores=NC, num_subcores=NS)
        fn = pl.kernel(body, out_type=out_type, mesh=mesh,
                       scratch_types=scratch_types)
        _sc_cache[name] = fn
    return fn


def _deg_body(sd_hbm, ones_hbm, z1_hbm, out_hbm, dstv, onesv, acc, sem):
    c = lax.axis_index("c")
    s = lax.axis_index("s")
    wid = c * NS + s
    pltpu.sync_copy(z1_hbm, acc.at[pl.ds(s * RPT, RPT)])
    pltpu.sync_copy(ones_hbm, onesv)
    plsc.subcore_barrier()

    def body(j, _):
        pltpu.sync_copy(sd_hbm.at[wid * NCHE + j, 1], dstv)
        pltpu.sync_copy(onesv, acc.at[dstv], add=True)
        return 0

    lax.fori_loop(0, NCHE, body, 0)
    plsc.subcore_barrier()
    pltpu.sync_copy(acc.at[pl.ds(s * RPT, RPT)],
                    out_hbm.at[pl.ds(c * NACC + s * RPT, RPT)])


def _deg_call(*args):
    return _sc_kernel(
        "deg", _deg_body,
        jax.ShapeDtypeStruct((NC * NACC,), jnp.float32),
        [
            pltpu.VMEM((ECH,), jnp.int32),
            pltpu.VMEM((ECH,), jnp.float32),
            pltpu.VMEM_SHARED((NACC,), jnp.float32),
            pltpu.SemaphoreType.DMA,
        ],
    )(*args)


def _agg_body(h_hbm, sd_hbm, z2_hbm, out_hbm, idx, r0, r1, r2, r3, acc,
              g0, g1, g2, g3, s0, s1, s2, s3,
              i0, i1, i2, i3, i4, i5, i6, i7):
    c = lax.axis_index("c")
    s = lax.axis_index("s")
    wid = c * NS + s
    base = wid * NCHE
    rows = (r0, r1, r2, r3)
    sg = (g0, g1, g2, g3)
    ss = (s0, s1, s2, s3)
    si = (i0, i1, i2, i3, i4, i5, i6, i7)

    pltpu.sync_copy(z2_hbm, acc.at[pl.ds(s * RPT, RPT)])
    plsc.subcore_barrier()

    for m in range(4):
        pltpu.async_copy(sd_hbm.at[base + m], idx.at[m], si[m])
    for j in range(2):
        pltpu.make_async_copy(sd_hbm.at[base + j], idx.at[j], si[j]).wait()
        pltpu.async_copy(h_hbm.at[idx.at[j, 0]], rows[j], sg[j])

    def body(t, _):
        for k in range(8):
            j = 8 * t + k
            b = k % 4
            b2 = (k + 2) % 4
            m2 = (k + 2) % 8
            m4 = (k + 4) % 8

            @pl.when(j >= 2)
            def _():
                pltpu.make_async_copy(
                    rows[b2], acc.at[idx.at[(k - 2) % 8, 1]], ss[b2]).wait()

            @pl.when(j + 2 < NCHE)
            def _():
                pltpu.make_async_copy(
                    sd_hbm.at[base + j + 2], idx.at[m2], si[m2]).wait()
                pltpu.async_copy(h_hbm.at[idx.at[m2, 0]], rows[b2], sg[b2])

            pltpu.make_async_copy(h_hbm.at[idx.at[k, 0]], rows[b], sg[b]).wait()
            pltpu.async_copy(rows[b], acc.at[idx.at[k, 1]], ss[b], add=True)

            @pl.when(j + 4 < NCHE)
            def _():
                pltpu.async_copy(sd_hbm.at[base + j + 4], idx.at[m4], si[m4])
        return 0

    lax.fori_loop(0, NCHE // 8, body, 0)
    pltpu.make_async_copy(rows[2], acc.at[idx.at[6, 1]], ss[2]).wait()
    pltpu.make_async_copy(rows[3], acc.at[idx.at[7, 1]], ss[3]).wait()
    plsc.subcore_barrier()
    pltpu.sync_copy(acc.at[pl.ds(s * RPT, RPT)], out_hbm.at[c, pl.ds(s * RPT, RPT)])


def _agg_call(*args):
    return _sc_kernel(
        "agg", _agg_body,
        jax.ShapeDtypeStruct((NC, NACC, D), jnp.float32),
        [
            pltpu.VMEM((8, 2, ECH), jnp.int32),
            pltpu.VMEM((ECH, D), jnp.float32),
            pltpu.VMEM((ECH, D), jnp.float32),
            pltpu.VMEM((ECH, D), jnp.float32),
            pltpu.VMEM((ECH, D), jnp.float32),
            pltpu.VMEM_SHARED((NACC, D), jnp.float32),
        ] + [pltpu.SemaphoreType.DMA] * 16,
    )(*args)


def _cgather_body(x_hbm, idx_hbm, out_hbm, idxv, rows, sem):
    c = lax.axis_index("c")
    s = lax.axis_index("s")
    wid = c * NS + s
    base = pl.multiple_of(wid * (B // NW), 8)
    pltpu.sync_copy(idx_hbm.at[pl.ds(base, B // NW)], idxv)
    pltpu.async_copy(x_hbm.at[idxv], rows, sem).wait()
    pltpu.sync_copy(rows, out_hbm.at[pl.ds(base, B // NW)])


def _cgather_call(*args):
    return _sc_kernel(
        "cgather", _cgather_body,
        jax.ShapeDtypeStruct((B, D), jnp.float32),
        [
            pltpu.VMEM((B // NW,), jnp.int32),
            pltpu.VMEM((B // NW, D), jnp.float32),
            pltpu.SemaphoreType.DMA,
        ],
    )(*args)


def _prep_body(d0, d1, x, w, dis_out, hh_out):
    dis = lax.rsqrt(1.0 + d0[...] + d1[...])
    dis_out[...] = dis
    hh_out[...] = dis * jnp.dot(x[...], w[...], preferred_element_type=jnp.float32)


def _prep(d0, d1, x, w):
    grid = N // RB
    return pl.pallas_call(
        _prep_body,
        grid=(grid,),
        in_specs=[
            pl.BlockSpec((RB, 1), lambda i: (i, 0)),
            pl.BlockSpec((RB, 1), lambda i: (i, 0)),
            pl.BlockSpec((RB, D), lambda i: (i, 0)),
            pl.BlockSpec((D, D), lambda i: (0, 0)),
        ],
        out_specs=[
            pl.BlockSpec((RB, 1), lambda i: (i, 0)),
            pl.BlockSpec((RB, D), lambda i: (i, 0)),
        ],
        out_shape=[
            jax.ShapeDtypeStruct((N, 1), jnp.float32),
            jax.ShapeDtypeStruct((N, D), jnp.float32),
        ],
    )(d0, d1, x, w)


def _stats_body(s0, s1, hh, dis, b, h_out, st_out):
    i = pl.program_id(0)
    h = dis[...] * (s0[0] + s1[0] + hh[...]) + b[...]
    h_out[...] = h

    @pl.when(i == 0)
    def _():
        st_out[...] = jnp.zeros_like(st_out)

    st_out[0:1, :] += jnp.sum(h, axis=0, keepdims=True)
    st_out[1:2, :] += jnp.sum(h * h, axis=0, keepdims=True)


def _stats(sp, hh, dis, b):
    grid = N // RB
    return pl.pallas_call(
        _stats_body,
        grid=(grid,),
        in_specs=[
            pl.BlockSpec((1, RB, D), lambda i: (0, i, 0)),
            pl.BlockSpec((1, RB, D), lambda i: (1, i, 0)),
            pl.BlockSpec((RB, D), lambda i: (i, 0)),
            pl.BlockSpec((RB, 1), lambda i: (i, 0)),
            pl.BlockSpec((1, D), lambda i: (0, 0)),
        ],
        out_specs=[
            pl.BlockSpec((RB, D), lambda i: (i, 0)),
            pl.BlockSpec((8, D), lambda i: (0, 0)),
        ],
        out_shape=[
            jax.ShapeDtypeStruct((N, D), jnp.float32),
            jax.ShapeDtypeStruct((8, D), jnp.float32),
        ],
    )(sp, sp, hh, dis, b)


def _bnmm_body(h, st, g, beta, dis, w, out):
    mu = st[0:1, :] * (1.0 / N)
    var = st[1:2, :] * (1.0 / N) - mu * mu
    sc = lax.rsqrt(var + EPS) * g[...]
    xn = jnp.maximum((h[...] - mu) * sc + beta[...], 0.0)
    out[...] = dis[...] * jnp.dot(xn, w[...], preferred_element_type=jnp.float32)


def _bnmm(h, st, g, beta, dis, w):
    grid = N // RB
    return pl.pallas_call(
        _bnmm_body,
        grid=(grid,),
        in_specs=[
            pl.BlockSpec((RB, D), lambda i: (i, 0)),
            pl.BlockSpec((8, D), lambda i: (0, 0)),
            pl.BlockSpec((1, D), lambda i: (0, 0)),
            pl.BlockSpec((1, D), lambda i: (0, 0)),
            pl.BlockSpec((RB, 1), lambda i: (i, 0)),
            pl.BlockSpec((D, D), lambda i: (0, 0)),
        ],
        out_specs=pl.BlockSpec((RB, D), lambda i: (i, 0)),
        out_shape=jax.ShapeDtypeStruct((N, D), jnp.float32),
    )(h, st, g, beta, dis, w)


def _bnfinal_body(h, st, g, beta, out):
    mu = st[0:1, :] * (1.0 / N)
    var = st[1:2, :] * (1.0 / N) - mu * mu
    sc = lax.rsqrt(var + EPS) * g[...]
    out[...] = jnp.maximum((h[...] - mu) * sc + beta[...], 0.0)


def _bnfinal(h, st, g, beta):
    grid = N // RB
    return pl.pallas_call(
        _bnfinal_body,
        grid=(grid,),
        in_specs=[
            pl.BlockSpec((RB, D), lambda i: (i, 0)),
            pl.BlockSpec((8, D), lambda i: (0, 0)),
            pl.BlockSpec((1, D), lambda i: (0, 0)),
            pl.BlockSpec((1, D), lambda i: (0, 0)),
        ],
        out_specs=pl.BlockSpec((RB, D), lambda i: (i, 0)),
        out_shape=jax.ShapeDtypeStruct((N, D), jnp.float32),
    )(h, st, g, beta)


def _tail_body(cid, xc, wfc, bfc, out):
    cid_ = cid[...]
    xc_ = xc[...]
    colsum = jnp.sum(cid_, axis=0, keepdims=True)
    cidn = cid_ / colsum
    cf = lax.dot_general(cidn, xc_, (((0,), (0,)), ((), ())),
                         preferred_element_type=jnp.float32)
    rmax = jnp.max(cid_, axis=1, keepdims=True)
    io = lax.broadcasted_iota(jnp.int32, (B, C), 1)
    am = jnp.min(jnp.where(cid_ == rmax, io, C), axis=1, keepdims=True)
    oh = (io == am).astype(jnp.float32)
    x1 = jnp.dot(oh, cf, preferred_element_type=jnp.float32)
    wt = wfc[0:D, :]
    wb = wfc[D:2 * D, :]
    bias = bfc[...]
    out[0:B, :] = (jnp.dot(xc_, wt, preferred_element_type=jnp.float32)
                   + jnp.dot(x1, wb, preferred_element_type=jnp.float32) + bias)
    out[B:2 * B, :] = (jnp.dot(x1, wt, preferred_element_type=jnp.float32)
                       + jnp.dot(xc_, wb, preferred_element_type=jnp.float32) + bias)


def _tail(cid, xc, wfc, bfc):
    return pl.pallas_call(
        _tail_body,
        out_shape=jax.ShapeDtypeStruct((2 * B, 2 * D), jnp.float32),
    )(cid, xc, wfc, bfc)


def kernel(x, edge_index, cluster_id, cluster_index,
           W1, b1, g1, beta1, W2, b2, g2, beta2, W3, b3, g3, beta3, Wfc, bfc):
    pad = EPAD - E
    pad_src = jnp.arange(pad, dtype=jnp.int32) % N
    src = jnp.concatenate([edge_index[0], pad_src])
    pad_dst = N + (jnp.arange(pad, dtype=jnp.int32) % (NACC - N))
    dst = jnp.concatenate([edge_index[1], pad_dst])
    sdp = jnp.stack([src.reshape(EPAD // ECH, ECH),
                     dst.reshape(EPAD // ECH, ECH)], axis=1)

    ones_ch = jnp.ones((ECH,), jnp.float32)
    z1 = jnp.zeros((RPT,), jnp.float32)
    z2 = jnp.zeros((RPT, D), jnp.float32)

    degp = _deg_call(sdp, ones_ch, z1).reshape(NC, NACC)
    d0 = degp[0, :N].reshape(N, 1)
    d1 = degp[1, :N].reshape(N, 1)
    dis, hh = _prep(d0, d1, x, W1)

    layers = ((b1, g1, beta1, W2), (b2, g2, beta2, W3), (b3, g3, beta3, None))
    xo = None
    for b, g, beta, wnext in layers:
        sp = _agg_call(hh, sdp, z2)
        h, st = _stats(sp, hh, dis, b.reshape(1, D))
        if wnext is not None:
            hh = _bnmm(h, st, g.reshape(1, D), beta.reshape(1, D), dis, wnext)
        else:
            xo = _bnfinal(h, st, g.reshape(1, D), beta.reshape(1, D))

    xc = _cgather_call(xo, cluster_index)
    return _tail(cluster_id, xc, Wfc, bfc)


# speedup vs baseline: 1.1234x; 1.1234x over previous
import functools

import jax
import jax.numpy as jnp
from jax import lax
from jax.experimental import pallas as pl
from jax.experimental.pallas import tpu as pltpu
from jax.experimental.pallas import tpu_sc as plsc

N = 10000
E = 320000
D = 128
B = 4096
C = 64
NC = 2
NS = 16
NW = NC * NS
ECH = 80
EPT = 10240
EPAD = EPT * NW
NCHE = EPT // ECH
CGC = B // NW
NACC = 10240
RPT = NACC // NS
RB = 1000
EPS = 1e-5

_sc_cache = {}


def _sc_kernel(name, body, out_type, scratch_types):
    fn = _sc_cache.get(name)
    if fn is None:
        mesh = plsc.VectorSubcoreMesh(core_axis_name="c", subcore_axis_name="s",
                                      num_cores=NC, num_subcores=NS)
        fn = pl.kernel(body, out_type=out_type, mesh=mesh,
                       scratch_types=scratch_types)
        _sc_cache[name] = fn
    return fn


DCH = 128
NCHD = EPT // DCH


def _deg_body(dst_hbm, ones_hbm, z1_hbm, out_hbm, d0, d1, onesv, acc,
              sem0, sem1):
    c = lax.axis_index("c")
    s = lax.axis_index("s")
    wid = c * NS + s
    base = wid * NCHD
    pltpu.sync_copy(z1_hbm.at[pl.ds(s * RPT, RPT)], acc.at[pl.ds(s * RPT, RPT)])
    pltpu.sync_copy(ones_hbm, onesv)
    plsc.subcore_barrier()

    pltpu.async_copy(dst_hbm.at[base], d0, sem0)

    def body(t, _):
        j0 = 2 * t
        pltpu.async_copy(dst_hbm.at[base + j0 + 1], d1, sem1)
        pltpu.make_async_copy(dst_hbm.at[base + j0], d0, sem0).wait()
        pltpu.sync_copy(onesv, acc.at[d0], add=True)

        @pl.when(t + 1 < NCHD // 2)
        def _():
            pltpu.async_copy(dst_hbm.at[base + j0 + 2], d0, sem0)

        pltpu.make_async_copy(dst_hbm.at[base + j0 + 1], d1, sem1).wait()
        pltpu.sync_copy(onesv, acc.at[d1], add=True)
        return 0

    lax.fori_loop(0, NCHD // 2, body, 0)
    plsc.subcore_barrier()
    pltpu.sync_copy(acc.at[pl.ds(s * RPT, RPT)],
                    out_hbm.at[pl.ds(c * NACC + s * RPT, RPT)])


def _deg_call(*args):
    return _sc_kernel(
        "deg", _deg_body,
        jax.ShapeDtypeStruct((NC * NACC,), jnp.float32),
        [
            pltpu.VMEM((DCH,), jnp.int32),
            pltpu.VMEM((DCH,), jnp.int32),
            pltpu.VMEM((DCH,), jnp.float32),
            pltpu.VMEM_SHARED((NACC,), jnp.float32),
            pltpu.SemaphoreType.DMA,
            pltpu.SemaphoreType.DMA,
        ],
    )(*args)


def _agg_body(h_hbm, sd_hbm, z2_hbm, out_hbm, idx, r0, r1, r2, r3, acc,
              g0, g1, g2, g3, s0, s1, s2, s3,
              i0, i1, i2, i3, i4, i5, i6, i7):
    c = lax.axis_index("c")
    s = lax.axis_index("s")
    wid = c * NS + s
    base = wid * NCHE
    rows = (r0, r1, r2, r3)
    sg = (g0, g1, g2, g3)
    ss = (s0, s1, s2, s3)
    si = (i0, i1, i2, i3, i4, i5, i6, i7)

    pltpu.sync_copy(z2_hbm.at[pl.ds(s * RPT, RPT)], acc.at[pl.ds(s * RPT, RPT)])
    plsc.subcore_barrier()

    for m in range(4):
        pltpu.async_copy(sd_hbm.at[base + m], idx.at[m], si[m])
    for j in range(2):
        pltpu.make_async_copy(sd_hbm.at[base + j], idx.at[j], si[j]).wait()
        pltpu.async_copy(h_hbm.at[idx.at[j, 0]], rows[j], sg[j])

    def body(t, _):
        for k in range(8):
            j = 8 * t + k
            b = k % 4
            b2 = (k + 2) % 4
            m2 = (k + 2) % 8
            m4 = (k + 4) % 8

            @pl.when(j >= 2)
            def _():
                pltpu.make_async_copy(
                    rows[b2], acc.at[idx.at[(k - 2) % 8, 1]], ss[b2]).wait()

            @pl.when(j + 2 < NCHE)
            def _():
                pltpu.make_async_copy(
                    sd_hbm.at[base + j + 2], idx.at[m2], si[m2]).wait()
                pltpu.async_copy(h_hbm.at[idx.at[m2, 0]], rows[b2], sg[b2])

            pltpu.make_async_copy(h_hbm.at[idx.at[k, 0]], rows[b], sg[b]).wait()
            pltpu.async_copy(rows[b], acc.at[idx.at[k, 1]], ss[b], add=True)

            @pl.when(j + 4 < NCHE)
            def _():
                pltpu.async_copy(sd_hbm.at[base + j + 4], idx.at[m4], si[m4])
        return 0

    lax.fori_loop(0, NCHE // 8, body, 0)
    pltpu.make_async_copy(rows[2], acc.at[idx.at[6, 1]], ss[2]).wait()
    pltpu.make_async_copy(rows[3], acc.at[idx.at[7, 1]], ss[3]).wait()
    plsc.subcore_barrier()
    pltpu.sync_copy(acc.at[pl.ds(s * RPT, RPT)], out_hbm.at[c, pl.ds(s * RPT, RPT)])


def _agg_call(*args):
    return _sc_kernel(
        "agg", _agg_body,
        jax.ShapeDtypeStruct((NC, NACC, D), jnp.float32),
        [
            pltpu.VMEM((8, 2, ECH), jnp.int32),
            pltpu.VMEM((ECH, D), jnp.float32),
            pltpu.VMEM((ECH, D), jnp.float32),
            pltpu.VMEM((ECH, D), jnp.float32),
            pltpu.VMEM((ECH, D), jnp.float32),
            pltpu.VMEM_SHARED((NACC, D), jnp.float32),
        ] + [pltpu.SemaphoreType.DMA] * 16,
    )(*args)


def _cgather_body(x_hbm, idx_hbm, out_hbm, idxv, rows, sem):
    c = lax.axis_index("c")
    s = lax.axis_index("s")
    wid = c * NS + s
    base = pl.multiple_of(wid * (B // NW), 8)
    pltpu.sync_copy(idx_hbm.at[pl.ds(base, B // NW)], idxv)
    pltpu.async_copy(x_hbm.at[idxv], rows, sem).wait()
    pltpu.sync_copy(rows, out_hbm.at[pl.ds(base, B // NW)])


def _cgather_call(*args):
    return _sc_kernel(
        "cgather", _cgather_body,
        jax.ShapeDtypeStruct((B, D), jnp.float32),
        [
            pltpu.VMEM((B // NW,), jnp.int32),
            pltpu.VMEM((B // NW, D), jnp.float32),
            pltpu.SemaphoreType.DMA,
        ],
    )(*args)


def _prep_body(d0, d1, x, w, dis_out, hh_out):
    dis = lax.rsqrt(1.0 + d0[...] + d1[...])
    dis_out[...] = dis
    hh_out[...] = dis * jnp.dot(x[...], w[...],
                                preferred_element_type=jnp.float32)


def _prep(d0, d1, x, w):
    grid = N // RB
    return pl.pallas_call(
        _prep_body,
        grid=(grid,),
        in_specs=[
            pl.BlockSpec((RB, 1), lambda i: (i, 0)),
            pl.BlockSpec((RB, 1), lambda i: (i, 0)),
            pl.BlockSpec((RB, D), lambda i: (i, 0)),
            pl.BlockSpec((D, D), lambda i: (0, 0)),
        ],
        out_specs=[
            pl.BlockSpec((RB, 1), lambda i: (i, 0)),
            pl.BlockSpec((RB, D), lambda i: (i, 0)),
        ],
        out_shape=[
            jax.ShapeDtypeStruct((N, 1), jnp.float32),
            jax.ShapeDtypeStruct((N, D), jnp.float32),
        ],
    )(d0, d1, x, w)


_G = N // RB


def _layer_body(s0, s1, hh, dis, b, g, beta, w, out, hbuf, st):
    i = pl.program_id(0)

    @pl.when(i == 0)
    def _():
        st[...] = jnp.zeros_like(st)

    @pl.when(i < _G)
    def _():
        h = dis[...] * (s0[0] + s1[0] + hh[...]) + b[...]
        hbuf[pl.ds(i * RB, RB), :] = h
        st[0:1, :] += jnp.sum(h, axis=0, keepdims=True)
        st[1:2, :] += jnp.sum(h * h, axis=0, keepdims=True)

    @pl.when(i >= _G)
    def _():
        mu = st[0:1, :] * (1.0 / N)
        var = st[1:2, :] * (1.0 / N) - mu * mu
        sc = lax.rsqrt(var + EPS) * g[...]
        h = hbuf[pl.ds((i - _G) * RB, RB), :]
        xn = jnp.maximum((h - mu) * sc + beta[...], 0.0)
        if w is not None:
            out[...] = dis[...] * jnp.dot(xn, w[...],
                                          preferred_element_type=jnp.float32)
        else:
            out[...] = xn


def _layer(sp, hh, dis, b, g, beta, w):
    pa = lambda i: (0, jnp.where(i < _G, i, 0), 0)
    pb = lambda i: (1, jnp.where(i < _G, i, 0), 0)
    ab = lambda i: (jnp.where(i < _G, i, 0), 0)
    both = lambda i: (i % _G, 0)
    zero = lambda i: (0, 0)
    in_specs = [
        pl.BlockSpec((1, RB, D), pa),
        pl.BlockSpec((1, RB, D), pb),
        pl.BlockSpec((RB, D), ab),
        pl.BlockSpec((RB, 1), both),
        pl.BlockSpec((1, D), zero),
        pl.BlockSpec((1, D), zero),
        pl.BlockSpec((1, D), zero),
    ]
    args = [sp, sp, hh, dis, b, g, beta]
    if w is not None:
        in_specs.append(pl.BlockSpec((D, D), zero))
        args.append(w)
        body = _layer_body
    else:
        body = (lambda s0, s1, hh_, dis_, b_, g_, beta_, out, hbuf, st:
                _layer_body(s0, s1, hh_, dis_, b_, g_, beta_, None, out, hbuf, st))
    return pl.pallas_call(
        body,
        grid=(2 * _G,),
        in_specs=in_specs,
        out_specs=pl.BlockSpec((RB, D), lambda i: (jnp.maximum(i - _G, 0), 0)),
        out_shape=jax.ShapeDtypeStruct((N, D), jnp.float32),
        scratch_shapes=[
            pltpu.VMEM((N, D), jnp.float32),
            pltpu.VMEM((8, D), jnp.float32),
        ],
    )(*args)


def _tail_body(cid, xc, wfc, bfc, out):
    cid_ = cid[...]
    xc_ = xc[...]
    colsum = jnp.sum(cid_, axis=0, keepdims=True)
    cidn = cid_ / colsum
    cf = lax.dot_general(cidn, xc_, (((0,), (0,)), ((), ())),
                         preferred_element_type=jnp.float32)
    rmax = jnp.max(cid_, axis=1, keepdims=True)
    io = lax.broadcasted_iota(jnp.int32, (B, C), 1)
    am = jnp.min(jnp.where(cid_ == rmax, io, C), axis=1, keepdims=True)
    oh = (io == am).astype(jnp.float32)
    x1 = jnp.dot(oh, cf, preferred_element_type=jnp.float32)
    wt = wfc[0:D, :]
    wb = wfc[D:2 * D, :]
    bias = bfc[...]
    out[0:B, :] = (jnp.dot(xc_, wt, preferred_element_type=jnp.float32)
                   + jnp.dot(x1, wb, preferred_element_type=jnp.float32) + bias)
    out[B:2 * B, :] = (jnp.dot(x1, wt, preferred_element_type=jnp.float32)
                       + jnp.dot(xc_, wb, preferred_element_type=jnp.float32) + bias)


def _tail(cid, xc, wfc, bfc):
    return pl.pallas_call(
        _tail_body,
        out_shape=jax.ShapeDtypeStruct((2 * B, 2 * D), jnp.float32),
    )(cid, xc, wfc, bfc)


def kernel(x, edge_index, cluster_id, cluster_index,
           W1, b1, g1, beta1, W2, b2, g2, beta2, W3, b3, g3, beta3, Wfc, bfc):
    pad = EPAD - E
    pad_src = jnp.arange(pad, dtype=jnp.int32) % N
    src = jnp.concatenate([edge_index[0], pad_src])
    pad_dst = N + (jnp.arange(pad, dtype=jnp.int32) % (NACC - N))
    dst = jnp.concatenate([edge_index[1], pad_dst])
    sdp = jnp.stack([src.reshape(EPAD // ECH, ECH),
                     dst.reshape(EPAD // ECH, ECH)], axis=1)
    dstp = dst.reshape(EPAD // DCH, DCH)

    ones_ch = jnp.ones((DCH,), jnp.float32)
    z1 = jnp.zeros((NACC,), jnp.float32)
    z2 = jnp.zeros((NACC, D), jnp.float32)

    degp = _deg_call(dstp, ones_ch, z1).reshape(NC, NACC)
    d0 = degp[0, :N].reshape(N, 1)
    d1 = degp[1, :N].reshape(N, 1)
    dis, hh = _prep(d0, d1, x, W1)

    layers = ((b1, g1, beta1, W2), (b2, g2, beta2, W3), (b3, g3, beta3, None))
    xo = None
    for b, g, beta, wnext in layers:
        sp = _agg_call(hh, sdp, z2)
        o = _layer(sp, hh, dis, b.reshape(1, D), g.reshape(1, D),
                   beta.reshape(1, D), wnext)
        if wnext is not None:
            hh = o
        else:
            xo = o

    xc = _cgather_call(xo, cluster_index)
    return _tail(cluster_id, xc, Wfc, bfc)


# speedup vs baseline: 1.1369x; 1.0120x over previous
import functools

import jax
import jax.numpy as jnp
from jax import lax
from jax.experimental import pallas as pl
from jax.experimental.pallas import tpu as pltpu
from jax.experimental.pallas import tpu_sc as plsc

N = 10000
E = 320000
D = 128
B = 4096
C = 64
NC = 2
NS = 16
NW = NC * NS
ECH = 80
EPT = 10240
EPAD = EPT * NW
NCHE = EPT // ECH
CGC = B // NW
NACC = 10240
RPT = NACC // NS
RB = 1000
EPS = 1e-5

_sc_cache = {}


def _sc_kernel(name, body, out_type, scratch_types):
    fn = _sc_cache.get(name)
    if fn is None:
        mesh = plsc.VectorSubcoreMesh(core_axis_name="c", subcore_axis_name="s",
                                      num_cores=NC, num_subcores=NS)
        fn = pl.kernel(body, out_type=out_type, mesh=mesh,
                       scratch_types=scratch_types)
        _sc_cache[name] = fn
    return fn


DCH = 128
NCHD = EPT // DCH


def _deg_body(dst_hbm, ones_hbm, z1_hbm, out0_hbm, out1_hbm, d0, d1, onesv, acc,
              sem0, sem1):
    c = lax.axis_index("c")
    s = lax.axis_index("s")
    wid = c * NS + s
    base = wid * NCHD
    pltpu.sync_copy(z1_hbm.at[pl.ds(s * RPT, RPT)], acc.at[pl.ds(s * RPT, RPT)])
    pltpu.sync_copy(ones_hbm, onesv)
    plsc.subcore_barrier()

    pltpu.async_copy(dst_hbm.at[base], d0, sem0)

    def body(t, _):
        j0 = 2 * t
        pltpu.async_copy(dst_hbm.at[base + j0 + 1], d1, sem1)
        pltpu.make_async_copy(dst_hbm.at[base + j0], d0, sem0).wait()
        pltpu.sync_copy(onesv, acc.at[d0], add=True)

        @pl.when(t + 1 < NCHD // 2)
        def _():
            pltpu.async_copy(dst_hbm.at[base + j0 + 2], d0, sem0)

        pltpu.make_async_copy(dst_hbm.at[base + j0 + 1], d1, sem1).wait()
        pltpu.sync_copy(onesv, acc.at[d1], add=True)
        return 0

    lax.fori_loop(0, NCHD // 2, body, 0)
    plsc.subcore_barrier()

    @pl.when(c == 0)
    def _():
        pltpu.sync_copy(acc.at[pl.ds(s * RPT, RPT)],
                        out0_hbm.at[pl.ds(s * RPT, RPT)])

    @pl.when(c == 1)
    def _():
        pltpu.sync_copy(acc.at[pl.ds(s * RPT, RPT)],
                        out1_hbm.at[pl.ds(s * RPT, RPT)])


def _deg_call(*args):
    return _sc_kernel(
        "deg", _deg_body,
        [jax.ShapeDtypeStruct((NACC,), jnp.float32),
         jax.ShapeDtypeStruct((NACC,), jnp.float32)],
        [
            pltpu.VMEM((DCH,), jnp.int32),
            pltpu.VMEM((DCH,), jnp.int32),
            pltpu.VMEM((DCH,), jnp.float32),
            pltpu.VMEM_SHARED((NACC,), jnp.float32),
            pltpu.SemaphoreType.DMA,
            pltpu.SemaphoreType.DMA,
        ],
    )(*args)


def _agg_body(h_hbm, src_hbm, dst_hbm, z2_hbm, out_hbm, idx, r0, r1, r2, r3, acc,
              g0, g1, g2, g3, s0, s1, s2, s3,
              i0, i1, i2, i3, i4, i5, i6, i7):
    c = lax.axis_index("c")
    s = lax.axis_index("s")
    wid = c * NS + s
    base = wid * NCHE
    rows = (r0, r1, r2, r3)
    sg = (g0, g1, g2, g3)
    ss = (s0, s1, s2, s3)
    si = (i0, i1, i2, i3, i4, i5, i6, i7)

    pltpu.sync_copy(z2_hbm.at[pl.ds(s * RPT, RPT)], acc.at[pl.ds(s * RPT, RPT)])
    plsc.subcore_barrier()

    for m in range(4):
        pltpu.async_copy(src_hbm.at[base + m], idx.at[m, 0], si[m])
        pltpu.async_copy(dst_hbm.at[base + m], idx.at[m, 1], si[m])
    for j in range(2):
        pltpu.make_async_copy(src_hbm.at[base + j], idx.at[j, 0], si[j]).wait()
        pltpu.make_async_copy(dst_hbm.at[base + j], idx.at[j, 1], si[j]).wait()
        pltpu.async_copy(h_hbm.at[idx.at[j, 0]], rows[j], sg[j])

    def body(t, _):
        for k in range(8):
            j = 8 * t + k
            b = k % 4
            b2 = (k + 2) % 4
            m2 = (k + 2) % 8
            m4 = (k + 4) % 8

            @pl.when(j >= 2)
            def _():
                pltpu.make_async_copy(
                    rows[b2], acc.at[idx.at[(k - 2) % 8, 1]], ss[b2]).wait()

            @pl.when(j + 2 < NCHE)
            def _():
                pltpu.make_async_copy(
                    src_hbm.at[base + j + 2], idx.at[m2, 0], si[m2]).wait()
                pltpu.make_async_copy(
                    dst_hbm.at[base + j + 2], idx.at[m2, 1], si[m2]).wait()
                pltpu.async_copy(h_hbm.at[idx.at[m2, 0]], rows[b2], sg[b2])

            pltpu.make_async_copy(h_hbm.at[idx.at[k, 0]], rows[b], sg[b]).wait()
            pltpu.async_copy(rows[b], acc.at[idx.at[k, 1]], ss[b], add=True)

            @pl.when(j + 4 < NCHE)
            def _():
                pltpu.async_copy(src_hbm.at[base + j + 4], idx.at[m4, 0], si[m4])
                pltpu.async_copy(dst_hbm.at[base + j + 4], idx.at[m4, 1], si[m4])
        return 0

    lax.fori_loop(0, NCHE // 8, body, 0)
    pltpu.make_async_copy(rows[2], acc.at[idx.at[6, 1]], ss[2]).wait()
    pltpu.make_async_copy(rows[3], acc.at[idx.at[7, 1]], ss[3]).wait()
    plsc.subcore_barrier()
    pltpu.sync_copy(acc.at[pl.ds(s * RPT, RPT)], out_hbm.at[c, pl.ds(s * RPT, RPT)])


def _agg_call(*args):
    return _sc_kernel(
        "agg", _agg_body,
        jax.ShapeDtypeStruct((NC, NACC, D), jnp.float32),
        [
            pltpu.VMEM((8, 2, ECH), jnp.int32),
            pltpu.VMEM((ECH, D), jnp.float32),
            pltpu.VMEM((ECH, D), jnp.float32),
            pltpu.VMEM((ECH, D), jnp.float32),
            pltpu.VMEM((ECH, D), jnp.float32),
            pltpu.VMEM_SHARED((NACC, D), jnp.float32),
        ] + [pltpu.SemaphoreType.DMA] * 16,
    )(*args)


def _cgather_body(x_hbm, idx_hbm, out_hbm, idxv, rows, sem):
    c = lax.axis_index("c")
    s = lax.axis_index("s")
    wid = c * NS + s
    base = pl.multiple_of(wid * (B // NW), 8)
    pltpu.sync_copy(idx_hbm.at[pl.ds(base, B // NW)], idxv)
    pltpu.async_copy(x_hbm.at[idxv], rows, sem).wait()
    pltpu.sync_copy(rows, out_hbm.at[pl.ds(base, B // NW)])


def _cgather_call(*args):
    return _sc_kernel(
        "cgather", _cgather_body,
        jax.ShapeDtypeStruct((B, D), jnp.float32),
        [
            pltpu.VMEM((B // NW,), jnp.int32),
            pltpu.VMEM((B // NW, D), jnp.float32),
            pltpu.SemaphoreType.DMA,
        ],
    )(*args)


def _prep_body(d0, d1, x, w, dis_out, hh_out):
    dis = lax.rsqrt(1.0 + d0[...] + d1[...])
    dis_out[...] = dis
    hh_out[...] = dis * jnp.dot(x[...], w[...],
                                preferred_element_type=jnp.float32)


def _prep(d0, d1, x, w):
    grid = N // RB
    return pl.pallas_call(
        _prep_body,
        grid=(grid,),
        in_specs=[
            pl.BlockSpec((RB, 1), lambda i: (i, 0)),
            pl.BlockSpec((RB, 1), lambda i: (i, 0)),
            pl.BlockSpec((RB, D), lambda i: (i, 0)),
            pl.BlockSpec((D, D), lambda i: (0, 0)),
        ],
        out_specs=[
            pl.BlockSpec((RB, 1), lambda i: (i, 0)),
            pl.BlockSpec((RB, D), lambda i: (i, 0)),
        ],
        out_shape=[
            jax.ShapeDtypeStruct((N, 1), jnp.float32),
            jax.ShapeDtypeStruct((N, D), jnp.float32),
        ],
    )(d0, d1, x, w)


_G = N // RB


def _layer_body(s0, s1, hh, dis, b, g, beta, w, out, hbuf, st):
    i = pl.program_id(0)

    @pl.when(i == 0)
    def _():
        st[...] = jnp.zeros_like(st)

    @pl.when(i < _G)
    def _():
        h = dis[...] * (s0[0] + s1[0] + hh[...]) + b[...]
        hbuf[pl.ds(i * RB, RB), :] = h
        st[0:1, :] += jnp.sum(h, axis=0, keepdims=True)
        st[1:2, :] += jnp.sum(h * h, axis=0, keepdims=True)

    @pl.when(i >= _G)
    def _():
        mu = st[0:1, :] * (1.0 / N)
        var = st[1:2, :] * (1.0 / N) - mu * mu
        sc = lax.rsqrt(var + EPS) * g[...]
        h = hbuf[pl.ds((i - _G) * RB, RB), :]
        xn = jnp.maximum((h - mu) * sc + beta[...], 0.0)
        if w is not None:
            out[...] = dis[...] * jnp.dot(xn, w[...],
                                          preferred_element_type=jnp.float32)
        else:
            out[...] = xn


def _layer(sp, hh, dis, b, g, beta, w):
    pa = lambda i: (0, jnp.where(i < _G, i, 0), 0)
    pb = lambda i: (1, jnp.where(i < _G, i, 0), 0)
    ab = lambda i: (jnp.where(i < _G, i, 0), 0)
    both = lambda i: (i % _G, 0)
    zero = lambda i: (0, 0)
    in_specs = [
        pl.BlockSpec((1, RB, D), pa),
        pl.BlockSpec((1, RB, D), pb),
        pl.BlockSpec((RB, D), ab),
        pl.BlockSpec((RB, 1), both),
        pl.BlockSpec((1, D), zero),
        pl.BlockSpec((1, D), zero),
        pl.BlockSpec((1, D), zero),
    ]
    args = [sp, sp, hh, dis, b, g, beta]
    if w is not None:
        in_specs.append(pl.BlockSpec((D, D), zero))
        args.append(w)
        body = _layer_body
    else:
        body = (lambda s0, s1, hh_, dis_, b_, g_, beta_, out, hbuf, st:
                _layer_body(s0, s1, hh_, dis_, b_, g_, beta_, None, out, hbuf, st))
    return pl.pallas_call(
        body,
        grid=(2 * _G,),
        in_specs=in_specs,
        out_specs=pl.BlockSpec((RB, D), lambda i: (jnp.maximum(i - _G, 0), 0)),
        out_shape=jax.ShapeDtypeStruct((N, D), jnp.float32),
        scratch_shapes=[
            pltpu.VMEM((N, D), jnp.float32),
            pltpu.VMEM((8, D), jnp.float32),
        ],
    )(*args)


def _tail_body(cid, xc, wfc, bfc, out):
    cid_ = cid[...]
    xc_ = xc[...]
    colsum = jnp.sum(cid_, axis=0, keepdims=True)
    cidn = cid_ / colsum
    cf = lax.dot_general(cidn, xc_, (((0,), (0,)), ((), ())),
                         preferred_element_type=jnp.float32)
    rmax = jnp.max(cid_, axis=1, keepdims=True)
    io = lax.broadcasted_iota(jnp.int32, (B, C), 1)
    am = jnp.min(jnp.where(cid_ == rmax, io, C), axis=1, keepdims=True)
    oh = (io == am).astype(jnp.float32)
    x1 = jnp.dot(oh, cf, preferred_element_type=jnp.float32)
    wt = wfc[0:D, :]
    wb = wfc[D:2 * D, :]
    bias = bfc[...]
    out[0:B, :] = (jnp.dot(xc_, wt, preferred_element_type=jnp.float32)
                   + jnp.dot(x1, wb, preferred_element_type=jnp.float32) + bias)
    out[B:2 * B, :] = (jnp.dot(x1, wt, preferred_element_type=jnp.float32)
                       + jnp.dot(xc_, wb, preferred_element_type=jnp.float32) + bias)


def _tail(cid, xc, wfc, bfc):
    return pl.pallas_call(
        _tail_body,
        out_shape=jax.ShapeDtypeStruct((2 * B, 2 * D), jnp.float32),
    )(cid, xc, wfc, bfc)


def kernel(x, edge_index, cluster_id, cluster_index,
           W1, b1, g1, beta1, W2, b2, g2, beta2, W3, b3, g3, beta3, Wfc, bfc):
    pad = EPAD - E
    pad_src = jnp.arange(pad, dtype=jnp.int32) % N
    src = jnp.concatenate([edge_index[0], pad_src])
    pad_dst = N + (jnp.arange(pad, dtype=jnp.int32) % (NACC - N))
    dst = jnp.concatenate([edge_index[1], pad_dst])
    srcp = src.reshape(EPAD // ECH, ECH)
    dstpe = dst.reshape(EPAD // ECH, ECH)
    dstp = dst.reshape(EPAD // DCH, DCH)

    ones_ch = jnp.ones((DCH,), jnp.float32)
    z1 = jnp.zeros((NACC,), jnp.float32)
    z2 = jnp.zeros((NACC, D), jnp.float32)

    deg0, deg1 = _deg_call(dstp, ones_ch, z1)
    dis, hh = _prep(deg0.reshape(NACC, 1), deg1.reshape(NACC, 1), x, W1)

    layers = ((b1, g1, beta1, W2), (b2, g2, beta2, W3), (b3, g3, beta3, None))
    xo = None
    for b, g, beta, wnext in layers:
        sp = _agg_call(hh, srcp, dstpe, z2)
        o = _layer(sp, hh, dis, b.reshape(1, D), g.reshape(1, D),
                   beta.reshape(1, D), wnext)
        if wnext is not None:
            hh = o
        else:
            xo = o

    xc = _cgather_call(xo, cluster_index)
    return _tail(cluster_id, xc, Wfc, bfc)


# speedup vs baseline: 1.1753x; 1.0338x over previous
import functools

import jax
import jax.numpy as jnp
from jax import lax
from jax.experimental import pallas as pl
from jax.experimental.pallas import tpu as pltpu
from jax.experimental.pallas import tpu_sc as plsc

N = 10000
E = 320000
D = 128
B = 4096
C = 64
NC = 2
NS = 16
NW = NC * NS
ECH = 80
EPT = 10240
EPAD = EPT * NW
NCHE = EPT // ECH
CGC = B // NW
NACC = 10240
RPT = NACC // NS
RB = 2000
EPS = 1e-5

_sc_cache = {}


def _sc_kernel(name, body, out_type, scratch_types):
    fn = _sc_cache.get(name)
    if fn is None:
        mesh = plsc.VectorSubcoreMesh(core_axis_name="c", subcore_axis_name="s",
                                      num_cores=NC, num_subcores=NS)
        fn = pl.kernel(body, out_type=out_type, mesh=mesh,
                       scratch_types=scratch_types)
        _sc_cache[name] = fn
    return fn


DCH = 128
NCHD = EPT // DCH


def _deg_body(dst_hbm, ones_hbm, z1_hbm, out0_hbm, out1_hbm, d0, d1, onesv, acc,
              sem0, sem1):
    c = lax.axis_index("c")
    s = lax.axis_index("s")
    wid = c * NS + s
    base = wid * NCHD
    pltpu.sync_copy(z1_hbm.at[pl.ds(s * RPT, RPT)], acc.at[pl.ds(s * RPT, RPT)])
    pltpu.sync_copy(ones_hbm, onesv)
    plsc.subcore_barrier()

    pltpu.async_copy(dst_hbm.at[base], d0, sem0)

    def body(t, _):
        j0 = 2 * t
        pltpu.async_copy(dst_hbm.at[base + j0 + 1], d1, sem1)
        pltpu.make_async_copy(dst_hbm.at[base + j0], d0, sem0).wait()
        pltpu.sync_copy(onesv, acc.at[d0], add=True)

        @pl.when(t + 1 < NCHD // 2)
        def _():
            pltpu.async_copy(dst_hbm.at[base + j0 + 2], d0, sem0)

        pltpu.make_async_copy(dst_hbm.at[base + j0 + 1], d1, sem1).wait()
        pltpu.sync_copy(onesv, acc.at[d1], add=True)
        return 0

    lax.fori_loop(0, NCHD // 2, body, 0)
    plsc.subcore_barrier()

    @pl.when(c == 0)
    def _():
        pltpu.sync_copy(acc.at[pl.ds(s * RPT, RPT)],
                        out0_hbm.at[pl.ds(s * RPT, RPT)])

    @pl.when(c == 1)
    def _():
        pltpu.sync_copy(acc.at[pl.ds(s * RPT, RPT)],
                        out1_hbm.at[pl.ds(s * RPT, RPT)])


def _deg_call(*args):
    return _sc_kernel(
        "deg", _deg_body,
        [jax.ShapeDtypeStruct((NACC,), jnp.float32),
         jax.ShapeDtypeStruct((NACC,), jnp.float32)],
        [
            pltpu.VMEM((DCH,), jnp.int32),
            pltpu.VMEM((DCH,), jnp.int32),
            pltpu.VMEM((DCH,), jnp.float32),
            pltpu.VMEM_SHARED((NACC,), jnp.float32),
            pltpu.SemaphoreType.DMA,
            pltpu.SemaphoreType.DMA,
        ],
    )(*args)


def _agg_body(h_hbm, src_hbm, dst_hbm, z2_hbm, out_hbm, idx, r0, r1, r2, r3, acc,
              g0, g1, g2, g3, s0, s1, s2, s3,
              i0, i1, i2, i3, i4, i5, i6, i7):
    c = lax.axis_index("c")
    s = lax.axis_index("s")
    wid = c * NS + s
    base = wid * NCHE
    rows = (r0, r1, r2, r3)
    sg = (g0, g1, g2, g3)
    ss = (s0, s1, s2, s3)
    si = (i0, i1, i2, i3, i4, i5, i6, i7)

    pltpu.sync_copy(z2_hbm.at[pl.ds(s * RPT, RPT)], acc.at[pl.ds(s * RPT, RPT)])
    plsc.subcore_barrier()

    for m in range(4):
        pltpu.async_copy(src_hbm.at[base + m], idx.at[m, 0], si[m])
        pltpu.async_copy(dst_hbm.at[base + m], idx.at[m, 1], si[m])
    for j in range(2):
        pltpu.make_async_copy(src_hbm.at[base + j], idx.at[j, 0], si[j]).wait()
        pltpu.make_async_copy(dst_hbm.at[base + j], idx.at[j, 1], si[j]).wait()
        pltpu.async_copy(h_hbm.at[idx.at[j, 0]], rows[j], sg[j])

    def body(t, _):
        for k in range(8):
            j = 8 * t + k
            b = k % 4
            b2 = (k + 2) % 4
            m2 = (k + 2) % 8
            m4 = (k + 4) % 8

            @pl.when(j >= 2)
            def _():
                pltpu.make_async_copy(
                    rows[b2], acc.at[idx.at[(k - 2) % 8, 1]], ss[b2]).wait()

            @pl.when(j + 2 < NCHE)
            def _():
                pltpu.make_async_copy(
                    src_hbm.at[base + j + 2], idx.at[m2, 0], si[m2]).wait()
                pltpu.make_async_copy(
                    dst_hbm.at[base + j + 2], idx.at[m2, 1], si[m2]).wait()
                pltpu.async_copy(h_hbm.at[idx.at[m2, 0]], rows[b2], sg[b2])

            pltpu.make_async_copy(h_hbm.at[idx.at[k, 0]], rows[b], sg[b]).wait()
            pltpu.async_copy(rows[b], acc.at[idx.at[k, 1]], ss[b], add=True)

            @pl.when(j + 4 < NCHE)
            def _():
                pltpu.async_copy(src_hbm.at[base + j + 4], idx.at[m4, 0], si[m4])
                pltpu.async_copy(dst_hbm.at[base + j + 4], idx.at[m4, 1], si[m4])
        return 0

    lax.fori_loop(0, NCHE // 8, body, 0)
    pltpu.make_async_copy(rows[2], acc.at[idx.at[6, 1]], ss[2]).wait()
    pltpu.make_async_copy(rows[3], acc.at[idx.at[7, 1]], ss[3]).wait()
    plsc.subcore_barrier()
    pltpu.sync_copy(acc.at[pl.ds(s * RPT, RPT)], out_hbm.at[c, pl.ds(s * RPT, RPT)])


def _agg_call(*args):
    return _sc_kernel(
        "agg", _agg_body,
        jax.ShapeDtypeStruct((NC, NACC, D), jnp.float32),
        [
            pltpu.VMEM((8, 2, ECH), jnp.int32),
            pltpu.VMEM((ECH, D), jnp.float32),
            pltpu.VMEM((ECH, D), jnp.float32),
            pltpu.VMEM((ECH, D), jnp.float32),
            pltpu.VMEM((ECH, D), jnp.float32),
            pltpu.VMEM_SHARED((NACC, D), jnp.float32),
        ] + [pltpu.SemaphoreType.DMA] * 16,
    )(*args)


def _cgather_body(x_hbm, idx_hbm, out_hbm, idxv, rows, sem):
    c = lax.axis_index("c")
    s = lax.axis_index("s")
    wid = c * NS + s
    base = pl.multiple_of(wid * (B // NW), 8)
    pltpu.sync_copy(idx_hbm.at[pl.ds(base, B // NW)], idxv)
    pltpu.async_copy(x_hbm.at[idxv], rows, sem).wait()
    pltpu.sync_copy(rows, out_hbm.at[pl.ds(base, B // NW)])


def _cgather_call(*args):
    return _sc_kernel(
        "cgather", _cgather_body,
        jax.ShapeDtypeStruct((B, D), jnp.float32),
        [
            pltpu.VMEM((B // NW,), jnp.int32),
            pltpu.VMEM((B // NW, D), jnp.float32),
            pltpu.SemaphoreType.DMA,
        ],
    )(*args)


def _prep_body(d0, d1, x, w, dis_out, hh_out):
    dis = lax.rsqrt(1.0 + d0[...] + d1[...])
    dis_out[...] = dis
    hh_out[...] = dis * jnp.dot(x[...], w[...],
                                preferred_element_type=jnp.float32)


def _prep(d0, d1, x, w):
    grid = N // RB
    return pl.pallas_call(
        _prep_body,
        grid=(grid,),
        in_specs=[
            pl.BlockSpec((RB, 1), lambda i: (i, 0)),
            pl.BlockSpec((RB, 1), lambda i: (i, 0)),
            pl.BlockSpec((RB, D), lambda i: (i, 0)),
            pl.BlockSpec((D, D), lambda i: (0, 0)),
        ],
        out_specs=[
            pl.BlockSpec((RB, 1), lambda i: (i, 0)),
            pl.BlockSpec((RB, D), lambda i: (i, 0)),
        ],
        out_shape=[
            jax.ShapeDtypeStruct((N, 1), jnp.float32),
            jax.ShapeDtypeStruct((N, D), jnp.float32),
        ],
    )(d0, d1, x, w)


_G = N // RB


def _layer_body(s0, s1, hh, dis, b, g, beta, w, out, hbuf, st):
    i = pl.program_id(0)

    @pl.when(i == 0)
    def _():
        st[...] = jnp.zeros_like(st)

    @pl.when(i < _G)
    def _():
        h = dis[...] * (s0[0] + s1[0] + hh[...]) + b[...]
        hbuf[pl.ds(i * RB, RB), :] = h
        st[0:1, :] += jnp.sum(h, axis=0, keepdims=True)
        st[1:2, :] += jnp.sum(h * h, axis=0, keepdims=True)

    @pl.when(i >= _G)
    def _():
        mu = st[0:1, :] * (1.0 / N)
        var = st[1:2, :] * (1.0 / N) - mu * mu
        sc = lax.rsqrt(var + EPS) * g[...]
        h = hbuf[pl.ds((i - _G) * RB, RB), :]
        xn = jnp.maximum((h - mu) * sc + beta[...], 0.0)
        if w is not None:
            out[...] = dis[...] * jnp.dot(xn, w[...],
                                          preferred_element_type=jnp.float32)
        else:
            out[...] = xn


def _layer(sp, hh, dis, b, g, beta, w):
    pa = lambda i: (0, jnp.where(i < _G, i, 0), 0)
    pb = lambda i: (1, jnp.where(i < _G, i, 0), 0)
    ab = lambda i: (jnp.where(i < _G, i, 0), 0)
    both = lambda i: (i % _G, 0)
    zero = lambda i: (0, 0)
    in_specs = [
        pl.BlockSpec((1, RB, D), pa),
        pl.BlockSpec((1, RB, D), pb),
        pl.BlockSpec((RB, D), ab),
        pl.BlockSpec((RB, 1), both),
        pl.BlockSpec((1, D), zero),
        pl.BlockSpec((1, D), zero),
        pl.BlockSpec((1, D), zero),
    ]
    args = [sp, sp, hh, dis, b, g, beta]
    if w is not None:
        in_specs.append(pl.BlockSpec((D, D), zero))
        args.append(w)
        body = _layer_body
    else:
        body = (lambda s0, s1, hh_, dis_, b_, g_, beta_, out, hbuf, st:
                _layer_body(s0, s1, hh_, dis_, b_, g_, beta_, None, out, hbuf, st))
    return pl.pallas_call(
        body,
        grid=(2 * _G,),
        in_specs=in_specs,
        out_specs=pl.BlockSpec((RB, D), lambda i: (jnp.maximum(i - _G, 0), 0)),
        out_shape=jax.ShapeDtypeStruct((N, D), jnp.float32),
        scratch_shapes=[
            pltpu.VMEM((N, D), jnp.float32),
            pltpu.VMEM((8, D), jnp.float32),
        ],
    )(*args)


def _tail_body(cid, xc, wfc, bfc, out):
    cid_ = cid[...]
    xc_ = xc[...]
    colsum = jnp.sum(cid_, axis=0, keepdims=True)
    cidn = cid_ / colsum
    cf = lax.dot_general(cidn, xc_, (((0,), (0,)), ((), ())),
                         preferred_element_type=jnp.float32)
    rmax = jnp.max(cid_, axis=1, keepdims=True)
    io = lax.broadcasted_iota(jnp.int32, (B, C), 1)
    am = jnp.min(jnp.where(cid_ == rmax, io, C), axis=1, keepdims=True)
    oh = (io == am).astype(jnp.float32)
    x1 = jnp.dot(oh, cf, preferred_element_type=jnp.float32)
    wt = wfc[0:D, :]
    wb = wfc[D:2 * D, :]
    bias = bfc[...]
    out[0:B, :] = (jnp.dot(xc_, wt, preferred_element_type=jnp.float32)
                   + jnp.dot(x1, wb, preferred_element_type=jnp.float32) + bias)
    out[B:2 * B, :] = (jnp.dot(x1, wt, preferred_element_type=jnp.float32)
                       + jnp.dot(xc_, wb, preferred_element_type=jnp.float32) + bias)


def _tail(cid, xc, wfc, bfc):
    return pl.pallas_call(
        _tail_body,
        out_shape=jax.ShapeDtypeStruct((2 * B, 2 * D), jnp.float32),
    )(cid, xc, wfc, bfc)


def kernel(x, edge_index, cluster_id, cluster_index,
           W1, b1, g1, beta1, W2, b2, g2, beta2, W3, b3, g3, beta3, Wfc, bfc):
    pad = EPAD - E
    pad_src = jnp.arange(pad, dtype=jnp.int32) % N
    src = jnp.concatenate([edge_index[0], pad_src])
    pad_dst = N + (jnp.arange(pad, dtype=jnp.int32) % (NACC - N))
    dst = jnp.concatenate([edge_index[1], pad_dst])
    srcp = src.reshape(EPAD // ECH, ECH)
    dstpe = dst.reshape(EPAD // ECH, ECH)
    dstp = dst.reshape(EPAD // DCH, DCH)

    ones_ch = jnp.ones((DCH,), jnp.float32)
    z1 = jnp.zeros((NACC,), jnp.float32)
    z2 = jnp.zeros((NACC, D), jnp.float32)

    deg0, deg1 = _deg_call(dstp, ones_ch, z1)
    dis, hh = _prep(deg0.reshape(NACC, 1), deg1.reshape(NACC, 1), x, W1)

    layers = ((b1, g1, beta1, W2), (b2, g2, beta2, W3), (b3, g3, beta3, None))
    xo = None
    for b, g, beta, wnext in layers:
        sp = _agg_call(hh, srcp, dstpe, z2)
        o = _layer(sp, hh, dis, b.reshape(1, D), g.reshape(1, D),
                   beta.reshape(1, D), wnext)
        if wnext is not None:
            hh = o
        else:
            xo = o

    xc = _cgather_call(xo, cluster_index)
    return _tail(cluster_id, xc, Wfc, bfc)


# speedup vs baseline: 1.1919x; 1.0142x over previous
import functools

import jax
import jax.numpy as jnp
from jax import lax
from jax.experimental import pallas as pl
from jax.experimental.pallas import tpu as pltpu
from jax.experimental.pallas import tpu_sc as plsc

N = 10000
E = 320000
D = 128
B = 4096
C = 64
NC = 2
NS = 16
NW = NC * NS
ECH = 80
EPT = 10240
EPAD = EPT * NW
NCHE = EPT // ECH
CGC = B // NW
NACC = 10240
RPT = NACC // NS
RB = 5000
EPS = 1e-5

_sc_cache = {}


def _sc_kernel(name, body, out_type, scratch_types):
    fn = _sc_cache.get(name)
    if fn is None:
        mesh = plsc.VectorSubcoreMesh(core_axis_name="c", subcore_axis_name="s",
                                      num_cores=NC, num_subcores=NS)
        fn = pl.kernel(body, out_type=out_type, mesh=mesh,
                       scratch_types=scratch_types)
        _sc_cache[name] = fn
    return fn


DCH = 128
NCHD = EPT // DCH


def _deg_body(dst_hbm, ones_hbm, z1_hbm, out0_hbm, out1_hbm, d0, d1, onesv, acc,
              sem0, sem1):
    c = lax.axis_index("c")
    s = lax.axis_index("s")
    wid = c * NS + s
    base = wid * NCHD
    pltpu.sync_copy(z1_hbm.at[pl.ds(s * RPT, RPT)], acc.at[pl.ds(s * RPT, RPT)])
    pltpu.sync_copy(ones_hbm, onesv)
    plsc.subcore_barrier()

    pltpu.async_copy(dst_hbm.at[base], d0, sem0)

    def body(t, _):
        j0 = 2 * t
        pltpu.async_copy(dst_hbm.at[base + j0 + 1], d1, sem1)
        pltpu.make_async_copy(dst_hbm.at[base + j0], d0, sem0).wait()
        pltpu.sync_copy(onesv, acc.at[d0], add=True)

        @pl.when(t + 1 < NCHD // 2)
        def _():
            pltpu.async_copy(dst_hbm.at[base + j0 + 2], d0, sem0)

        pltpu.make_async_copy(dst_hbm.at[base + j0 + 1], d1, sem1).wait()
        pltpu.sync_copy(onesv, acc.at[d1], add=True)
        return 0

    lax.fori_loop(0, NCHD // 2, body, 0)
    plsc.subcore_barrier()

    @pl.when(c == 0)
    def _():
        pltpu.sync_copy(acc.at[pl.ds(s * RPT, RPT)],
                        out0_hbm.at[pl.ds(s * RPT, RPT)])

    @pl.when(c == 1)
    def _():
        pltpu.sync_copy(acc.at[pl.ds(s * RPT, RPT)],
                        out1_hbm.at[pl.ds(s * RPT, RPT)])


def _deg_call(*args):
    return _sc_kernel(
        "deg", _deg_body,
        [jax.ShapeDtypeStruct((NACC,), jnp.float32),
         jax.ShapeDtypeStruct((NACC,), jnp.float32)],
        [
            pltpu.VMEM((DCH,), jnp.int32),
            pltpu.VMEM((DCH,), jnp.int32),
            pltpu.VMEM((DCH,), jnp.float32),
            pltpu.VMEM_SHARED((NACC,), jnp.float32),
            pltpu.SemaphoreType.DMA,
            pltpu.SemaphoreType.DMA,
        ],
    )(*args)


def _agg_body(h_hbm, src_hbm, dst_hbm, z2_hbm, out_hbm, idx, r0, r1, r2, r3, acc,
              g0, g1, g2, g3, s0, s1, s2, s3,
              i0, i1, i2, i3, i4, i5, i6, i7):
    c = lax.axis_index("c")
    s = lax.axis_index("s")
    wid = c * NS + s
    base = wid * NCHE
    rows = (r0, r1, r2, r3)
    sg = (g0, g1, g2, g3)
    ss = (s0, s1, s2, s3)
    si = (i0, i1, i2, i3, i4, i5, i6, i7)

    pltpu.sync_copy(z2_hbm.at[pl.ds(s * RPT, RPT)], acc.at[pl.ds(s * RPT, RPT)])
    plsc.subcore_barrier()

    for m in range(4):
        pltpu.async_copy(src_hbm.at[base + m], idx.at[m, 0], si[m])
        pltpu.async_copy(dst_hbm.at[base + m], idx.at[m, 1], si[m])
    for j in range(2):
        pltpu.make_async_copy(src_hbm.at[base + j], idx.at[j, 0], si[j]).wait()
        pltpu.make_async_copy(dst_hbm.at[base + j], idx.at[j, 1], si[j]).wait()
        pltpu.async_copy(h_hbm.at[idx.at[j, 0]], rows[j], sg[j])

    def body(t, _):
        for k in range(8):
            j = 8 * t + k
            b = k % 4
            b2 = (k + 2) % 4
            m2 = (k + 2) % 8
            m4 = (k + 4) % 8

            @pl.when(j >= 2)
            def _():
                pltpu.make_async_copy(
                    rows[b2], acc.at[idx.at[(k - 2) % 8, 1]], ss[b2]).wait()

            @pl.when(j + 2 < NCHE)
            def _():
                pltpu.make_async_copy(
                    src_hbm.at[base + j + 2], idx.at[m2, 0], si[m2]).wait()
                pltpu.make_async_copy(
                    dst_hbm.at[base + j + 2], idx.at[m2, 1], si[m2]).wait()
                pltpu.async_copy(h_hbm.at[idx.at[m2, 0]], rows[b2], sg[b2])

            pltpu.make_async_copy(h_hbm.at[idx.at[k, 0]], rows[b], sg[b]).wait()
            pltpu.async_copy(rows[b], acc.at[idx.at[k, 1]], ss[b], add=True)

            @pl.when(j + 4 < NCHE)
            def _():
                pltpu.async_copy(src_hbm.at[base + j + 4], idx.at[m4, 0], si[m4])
                pltpu.async_copy(dst_hbm.at[base + j + 4], idx.at[m4, 1], si[m4])
        return 0

    lax.fori_loop(0, NCHE // 8, body, 0)
    pltpu.make_async_copy(rows[2], acc.at[idx.at[6, 1]], ss[2]).wait()
    pltpu.make_async_copy(rows[3], acc.at[idx.at[7, 1]], ss[3]).wait()
    plsc.subcore_barrier()
    pltpu.sync_copy(acc.at[pl.ds(s * RPT, RPT)], out_hbm.at[c, pl.ds(s * RPT, RPT)])


def _agg_call(*args):
    return _sc_kernel(
        "agg", _agg_body,
        jax.ShapeDtypeStruct((NC, NACC, D), jnp.float32),
        [
            pltpu.VMEM((8, 2, ECH), jnp.int32),
            pltpu.VMEM((ECH, D), jnp.float32),
            pltpu.VMEM((ECH, D), jnp.float32),
            pltpu.VMEM((ECH, D), jnp.float32),
            pltpu.VMEM((ECH, D), jnp.float32),
            pltpu.VMEM_SHARED((NACC, D), jnp.float32),
        ] + [pltpu.SemaphoreType.DMA] * 16,
    )(*args)


def _cgather_body(x_hbm, idx_hbm, out_hbm, idxv, rows, sem):
    c = lax.axis_index("c")
    s = lax.axis_index("s")
    wid = c * NS + s
    base = pl.multiple_of(wid * (B // NW), 8)
    pltpu.sync_copy(idx_hbm.at[pl.ds(base, B // NW)], idxv)
    pltpu.async_copy(x_hbm.at[idxv], rows, sem).wait()
    pltpu.sync_copy(rows, out_hbm.at[pl.ds(base, B // NW)])


def _cgather_call(*args):
    return _sc_kernel(
        "cgather", _cgather_body,
        jax.ShapeDtypeStruct((B, D), jnp.float32),
        [
            pltpu.VMEM((B // NW,), jnp.int32),
            pltpu.VMEM((B // NW, D), jnp.float32),
            pltpu.SemaphoreType.DMA,
        ],
    )(*args)


def _prep_body(d0, d1, x, w, dis_out, hh_out):
    dis = lax.rsqrt(1.0 + d0[...] + d1[...])
    dis_out[...] = dis
    hh_out[...] = dis * jnp.dot(x[...], w[...],
                                preferred_element_type=jnp.float32)


def _prep(d0, d1, x, w):
    grid = N // RB
    return pl.pallas_call(
        _prep_body,
        grid=(grid,),
        in_specs=[
            pl.BlockSpec((RB, 1), lambda i: (i, 0)),
            pl.BlockSpec((RB, 1), lambda i: (i, 0)),
            pl.BlockSpec((RB, D), lambda i: (i, 0)),
            pl.BlockSpec((D, D), lambda i: (0, 0)),
        ],
        out_specs=[
            pl.BlockSpec((RB, 1), lambda i: (i, 0)),
            pl.BlockSpec((RB, D), lambda i: (i, 0)),
        ],
        out_shape=[
            jax.ShapeDtypeStruct((N, 1), jnp.float32),
            jax.ShapeDtypeStruct((N, D), jnp.float32),
        ],
    )(d0, d1, x, w)


_G = N // RB


def _layer_body(s0, s1, hh, dis, b, g, beta, w, out, hbuf, st):
    i = pl.program_id(0)

    @pl.when(i == 0)
    def _():
        st[...] = jnp.zeros_like(st)

    @pl.when(i < _G)
    def _():
        h = dis[...] * (s0[0] + s1[0] + hh[...]) + b[...]
        hbuf[pl.ds(i * RB, RB), :] = h
        st[0:1, :] += jnp.sum(h, axis=0, keepdims=True)
        st[1:2, :] += jnp.sum(h * h, axis=0, keepdims=True)

    @pl.when(i >= _G)
    def _():
        mu = st[0:1, :] * (1.0 / N)
        var = st[1:2, :] * (1.0 / N) - mu * mu
        sc = lax.rsqrt(var + EPS) * g[...]
        h = hbuf[pl.ds((i - _G) * RB, RB), :]
        xn = jnp.maximum((h - mu) * sc + beta[...], 0.0)
        if w is not None:
            out[...] = dis[...] * jnp.dot(xn, w[...],
                                          preferred_element_type=jnp.float32)
        else:
            out[...] = xn


def _layer(sp, hh, dis, b, g, beta, w):
    pa = lambda i: (0, jnp.where(i < _G, i, 0), 0)
    pb = lambda i: (1, jnp.where(i < _G, i, 0), 0)
    ab = lambda i: (jnp.where(i < _G, i, 0), 0)
    both = lambda i: (i % _G, 0)
    zero = lambda i: (0, 0)
    in_specs = [
        pl.BlockSpec((1, RB, D), pa),
        pl.BlockSpec((1, RB, D), pb),
        pl.BlockSpec((RB, D), ab),
        pl.BlockSpec((RB, 1), both),
        pl.BlockSpec((1, D), zero),
        pl.BlockSpec((1, D), zero),
        pl.BlockSpec((1, D), zero),
    ]
    args = [sp, sp, hh, dis, b, g, beta]
    if w is not None:
        in_specs.append(pl.BlockSpec((D, D), zero))
        args.append(w)
        body = _layer_body
    else:
        body = (lambda s0, s1, hh_, dis_, b_, g_, beta_, out, hbuf, st:
                _layer_body(s0, s1, hh_, dis_, b_, g_, beta_, None, out, hbuf, st))
    return pl.pallas_call(
        body,
        grid=(2 * _G,),
        in_specs=in_specs,
        out_specs=pl.BlockSpec((RB, D), lambda i: (jnp.maximum(i - _G, 0), 0)),
        out_shape=jax.ShapeDtypeStruct((N, D), jnp.float32),
        scratch_shapes=[
            pltpu.VMEM((N, D), jnp.float32),
            pltpu.VMEM((8, D), jnp.float32),
        ],
    )(*args)


def _tail_body(cid, xc, wfc, bfc, out):
    cid_ = cid[...]
    xc_ = xc[...]
    colsum = jnp.sum(cid_, axis=0, keepdims=True)
    cidn = cid_ / colsum
    cf = lax.dot_general(cidn, xc_, (((0,), (0,)), ((), ())),
                         preferred_element_type=jnp.float32)
    rmax = jnp.max(cid_, axis=1, keepdims=True)
    io = lax.broadcasted_iota(jnp.int32, (B, C), 1)
    am = jnp.min(jnp.where(cid_ == rmax, io, C), axis=1, keepdims=True)
    oh = (io == am).astype(jnp.float32)
    x1 = jnp.dot(oh, cf, preferred_element_type=jnp.float32)
    wt = wfc[0:D, :]
    wb = wfc[D:2 * D, :]
    bias = bfc[...]
    out[0:B, :] = (jnp.dot(xc_, wt, preferred_element_type=jnp.float32)
                   + jnp.dot(x1, wb, preferred_element_type=jnp.float32) + bias)
    out[B:2 * B, :] = (jnp.dot(x1, wt, preferred_element_type=jnp.float32)
                       + jnp.dot(xc_, wb, preferred_element_type=jnp.float32) + bias)


def _tail(cid, xc, wfc, bfc):
    return pl.pallas_call(
        _tail_body,
        out_shape=jax.ShapeDtypeStruct((2 * B, 2 * D), jnp.float32),
    )(cid, xc, wfc, bfc)


def kernel(x, edge_index, cluster_id, cluster_index,
           W1, b1, g1, beta1, W2, b2, g2, beta2, W3, b3, g3, beta3, Wfc, bfc):
    pad = EPAD - E
    pad_src = jnp.arange(pad, dtype=jnp.int32) % N
    src = jnp.concatenate([edge_index[0], pad_src])
    pad_dst = N + (jnp.arange(pad, dtype=jnp.int32) % (NACC - N))
    dst = jnp.concatenate([edge_index[1], pad_dst])
    srcp = src.reshape(EPAD // ECH, ECH)
    dstpe = dst.reshape(EPAD // ECH, ECH)
    dstp = dst.reshape(EPAD // DCH, DCH)

    ones_ch = jnp.ones((DCH,), jnp.float32)
    z1 = jnp.zeros((NACC,), jnp.float32)
    z2 = jnp.zeros((NACC, D), jnp.float32)

    deg0, deg1 = _deg_call(dstp, ones_ch, z1)
    dis, hh = _prep(deg0.reshape(NACC, 1), deg1.reshape(NACC, 1), x, W1)

    layers = ((b1, g1, beta1, W2), (b2, g2, beta2, W3), (b3, g3, beta3, None))
    xo = None
    for b, g, beta, wnext in layers:
        sp = _agg_call(hh, srcp, dstpe, z2)
        o = _layer(sp, hh, dis, b.reshape(1, D), g.reshape(1, D),
                   beta.reshape(1, D), wnext)
        if wnext is not None:
            hh = o
        else:
            xo = o

    xc = _cgather_call(xo, cluster_index)
    return _tail(cluster_id, xc, Wfc, bfc)
